# baseline (speedup 1.0000x reference)
import functools

import jax
import jax.numpy as jnp
from jax import lax
from jax.experimental import pallas as pl

D = 384
K = 30
T = 5
C = 5
P = 196
Q = 75


def _mlp_core(x, w):
    w1, b1, w2, b2, a = w
    h = jnp.dot(x, w1, preferred_element_type=jnp.float32) + b1
    h = jnp.where(h >= 0, h, a * h)
    return jnp.dot(h, w2, preferred_element_type=jnp.float32) + b2


def _topk_mask(vals, k):
    r, n = vals.shape
    iota = lax.broadcasted_iota(jnp.int32, (r, n), 1)

    v = vals
    m = jnp.zeros((r, n), jnp.float32)
    for _ in range(k):
        mx = jnp.max(v, axis=1, keepdims=True)
        elig = v == mx
        idx = jnp.min(jnp.where(elig, iota, n), axis=1, keepdims=True)
        onehot = iota == idx
        m = jnp.maximum(m, onehot.astype(jnp.float32))
        v = jnp.where(onehot, -jnp.inf, v)
    return m


def _enhance(cls, pats):
    R = T * C
    diff = pats - cls[:, None, :]
    dsq = (diff * diff).reshape(R * P, D)
    ones = jnp.ones((1, D), jnp.float32)
    d2row = lax.dot_general(ones, dsq, (((1,), (1,)), ((), ())),
                            preferred_element_type=jnp.float32)
    d2 = jnp.concatenate(
        [lax.slice(d2row, (0, r * P), (1, (r + 1) * P)) for r in range(R)],
        axis=0)
    dist = jnp.sqrt(jnp.maximum(d2, 0.0))

    dist3 = dist.reshape(T, C, P)
    total0 = jnp.sum(dist3[:, 0, :], axis=0)
    other = total0[None, :] - dist3[:, 0, :]
    sim = dist3 / (other[:, None, :] + 1e-6)

    mask = _topk_mask(sim.reshape(R, P), K)
    sel = lax.dot_general(mask[:, None, :], pats,
                          (((2,), (1,)), ((0,), (0,))),
                          preferred_element_type=jnp.float32)[:, 0, :]
    enh = 2.0 * cls + sel / float(K)
    return jnp.sum(enh.reshape(T, C, D), axis=1)


def _enh_body(s_ref, d_ref, eps_ref, dpe_ref,
              dw1_ref, db1_ref, dw2_ref, db2_ref, da_ref,
              pw1_ref, pb1_ref, pw2_ref, pb2_ref, pa_ref, o_ref):
    g = pl.program_id(0)
    pw = (pw1_ref[...], pb1_ref[...], pw2_ref[...], pb2_ref[...],
          pa_ref[0, 0])

    @pl.when(g == 0)
    def _support():
        x = eps_ref[...].reshape(T * C * P, D)
        pats = (x + _mlp_core(x, pw)).reshape(T * C, P, D)
        o_ref[0] = _enhance(s_ref[...], pats)

    @pl.when(g == 1)
    def _dalle():
        dw = (dw1_ref[...], db1_ref[...], dw2_ref[...], db2_ref[...],
              da_ref[0, 0])
        cls = _mlp_core(d_ref[...], dw)
        x = dpe_ref[...].reshape(T * C * P, D)
        pre = _mlp_core(x, dw)
        pats = (pre + _mlp_core(pre, pw)).reshape(T * C, P, D)
        o_ref[0] = _enhance(cls, pats)


def _run_enh(s, d_in, eps_s, dpe, da, pa):
    cspec2 = pl.BlockSpec((T * C, D), lambda i: (0, 0))
    cspec3 = pl.BlockSpec((T * C, P, D), lambda i: (0, 0, 0))
    wspec = [
        pl.BlockSpec((D, D), lambda i: (0, 0)),
        pl.BlockSpec((1, D), lambda i: (0, 0)),
        pl.BlockSpec((D, D), lambda i: (0, 0)),
        pl.BlockSpec((1, D), lambda i: (0, 0)),
        pl.BlockSpec((1, 1), lambda i: (0, 0)),
    ]
    return pl.pallas_call(
        _enh_body,
        grid=(2,),
        in_specs=[cspec2, cspec2, cspec3, cspec3] + wspec + wspec,
        out_specs=pl.BlockSpec((1, T, D), lambda i: (i, 0, 0)),
        out_shape=jax.ShapeDtypeStruct((2, T, D), jnp.float32),
    )(s, d_in, eps_s, dpe, *da, *pa)


def _fw_body(qb, s0_ref, s1_ref, pat_ref, q_ref, w1_ref, b1_ref, w2_ref,
             b2_ref, a_ref, o_ref):
    w = (w1_ref[...], b1_ref[...], w2_ref[...], b2_ref[...], a_ref[0, 0])
    protos = (s0_ref[...] + s1_ref[...]) / float(2 * C)
    x = pat_ref[...].reshape(qb * P, D)
    pf = x + _mlp_core(x, w)
    pats = pf.reshape(qb, P, D)
    qv = q_ref[0]

    dots = lax.dot_general(protos, pf, (((1,), (1,)), ((), ())),
                           preferred_element_type=jnp.float32)
    sq = pf * pf
    ones = jnp.ones((1, D), jnp.float32)
    pn = lax.dot_general(ones, sq, (((1,), (1,)), ((), ())),
                         preferred_element_type=jnp.float32)
    prn = jnp.sqrt(jnp.sum(protos * protos, axis=1, keepdims=True))
    cos = dots / jnp.maximum(jnp.sqrt(pn) * prn, 1e-8)

    rows = jnp.concatenate(
        [lax.slice(cos, (0, q * P), (T, (q + 1) * P)) for q in range(qb)],
        axis=0)
    mx = jnp.max(rows, axis=1, keepdims=True)
    e = jnp.exp(rows - mx)
    w_ = e / jnp.sum(e, axis=1, keepdims=True)
    wm = w_ * _topk_mask(w_, K)
    ws_all = lax.dot_general(wm.reshape(qb, T, P), pats,
                             (((2,), (1,)), ((0,), (0,))),
                             preferred_element_type=jnp.float32)
    for q in range(qb):
        o_ref[:, 0, q, :] = ws_all[q] + 2.0 * qv[q][None, :]


def _run_feature_walk(s0, s1, epq, qv, pa, qb):
    nb = Q // qb
    wspec = [
        pl.BlockSpec((D, D), lambda i: (0, 0)),
        pl.BlockSpec((1, D), lambda i: (0, 0)),
        pl.BlockSpec((D, D), lambda i: (0, 0)),
        pl.BlockSpec((1, D), lambda i: (0, 0)),
        pl.BlockSpec((1, 1), lambda i: (0, 0)),
    ]
    out = pl.pallas_call(
        functools.partial(_fw_body, qb),
        grid=(nb,),
        in_specs=[
            pl.BlockSpec((T, D), lambda i: (0, 0)),
            pl.BlockSpec((T, D), lambda i: (0, 0)),
            pl.BlockSpec((qb, P, D), lambda i: (i, 0, 0)),
            pl.BlockSpec((1, qb, D), lambda i: (i, 0, 0)),
        ] + wspec,
        out_specs=pl.BlockSpec((T, 1, qb, D), lambda i: (0, i, 0, 0)),
        out_shape=jax.ShapeDtypeStruct((T, nb, qb, D), jnp.float32),
    )(s0, s1, epq, qv.reshape(nb, qb, D), *pa)
    return out.reshape(T, Q, D)


def _fold_bn(p, pfx):
    s = 1.0 / jnp.sqrt(jnp.float32(1.0 + 1e-5))
    g1 = s * p[pfx + 'bn1_g']
    w1 = p[pfx + 'fc1_w'].T * g1[None, :]
    b1 = (p[pfx + 'fc1_b'] * g1 + p[pfx + 'bn1_b'])[None, :]
    g2 = s * p[pfx + 'bn2_g']
    w2 = p[pfx + 'fc2_w'].T * g2[None, :]
    b2 = (p[pfx + 'fc2_b'] * g2 + p[pfx + 'bn2_b'])[None, :]
    a = p[pfx + 'prelu'].reshape(1, 1)
    return (w1, b1, w2, b2, a)


def kernel(support_set_vectors, query_set_vectors, dalle_emb_support,
           emb_patch_support, emb_patch_query, dalle_patch_embedding, params):
    s = support_set_vectors.reshape(T * C, D)
    q = query_set_vectors.reshape(Q, D)
    d_in = dalle_emb_support.reshape(T * C, D)

    da = _fold_bn(params, 'da_')
    pa = _fold_bn(params, 'pa_')

    sums = _run_enh(s, d_in, emb_patch_support, dalle_patch_embedding,
                    da, pa)
    sum0, sum1 = sums[0], sums[1]
    protos = (sum0 + sum1) / float(2 * C)

    cls_ws = _run_feature_walk(sum0, sum1, emb_patch_query, q, pa, qb=25)
    return (protos, cls_ws)


# speedup vs baseline: 1.0544x; 1.0544x over previous
import functools

import jax
import jax.numpy as jnp
from jax import lax
from jax.experimental import pallas as pl

D = 384
K = 30
T = 5
C = 5
P = 196
Q = 75


def _mlp_core(x, w):
    w1, b1, w2, b2, a = w
    h = jnp.dot(x, w1, preferred_element_type=jnp.float32) + b1
    h = jnp.where(h >= 0, h, a * h)
    return jnp.dot(h, w2, preferred_element_type=jnp.float32) + b2


def _topk_mask(vals, k):
    r, n = vals.shape
    iota = lax.broadcasted_iota(jnp.int32, (r, n), 1)

    def step(_, carry):
        v, m = carry
        mx = jnp.max(v, axis=1, keepdims=True)
        elig = v == mx
        idx = jnp.min(jnp.where(elig, iota, n), axis=1, keepdims=True)
        onehot = iota == idx
        m = jnp.maximum(m, onehot.astype(jnp.float32))
        v = jnp.where(onehot, -jnp.inf, v)
        return v, m

    _, mask = lax.fori_loop(
        0, k, step, (vals, jnp.zeros((r, n), jnp.float32)))
    return mask


def _enhance(cls, pats):
    R = T * C
    diff = pats - cls[:, None, :]
    dsq = (diff * diff).reshape(R * P, D)
    ones = jnp.ones((1, D), jnp.float32)
    d2row = lax.dot_general(ones, dsq, (((1,), (1,)), ((), ())),
                            preferred_element_type=jnp.float32)
    d2 = jnp.concatenate(
        [lax.slice(d2row, (0, r * P), (1, (r + 1) * P)) for r in range(R)],
        axis=0)
    dist = jnp.sqrt(jnp.maximum(d2, 0.0))

    dist3 = dist.reshape(T, C, P)
    total0 = jnp.sum(dist3[:, 0, :], axis=0)
    other = total0[None, :] - dist3[:, 0, :]
    sim = dist3 / (other[:, None, :] + 1e-6)

    mask = _topk_mask(sim.reshape(R, P), K)
    sel = lax.dot_general(mask[:, None, :], pats,
                          (((2,), (1,)), ((0,), (0,))),
                          preferred_element_type=jnp.float32)[:, 0, :]
    enh = 2.0 * cls + sel / float(K)
    return jnp.sum(enh.reshape(T, C, D), axis=1)


def _enh_body(s_ref, d_ref, eps_ref, dpe_ref,
              dw1_ref, db1_ref, dw2_ref, db2_ref, da_ref,
              pw1_ref, pb1_ref, pw2_ref, pb2_ref, pa_ref, o_ref):
    g = pl.program_id(0)
    pw = (pw1_ref[...], pb1_ref[...], pw2_ref[...], pb2_ref[...],
          pa_ref[0, 0])

    @pl.when(g == 0)
    def _support():
        x = eps_ref[...].reshape(T * C * P, D)
        pats = (x + _mlp_core(x, pw)).reshape(T * C, P, D)
        o_ref[0] = _enhance(s_ref[...], pats)

    @pl.when(g == 1)
    def _dalle():
        dw = (dw1_ref[...], db1_ref[...], dw2_ref[...], db2_ref[...],
              da_ref[0, 0])
        cls = _mlp_core(d_ref[...], dw)
        x = dpe_ref[...].reshape(T * C * P, D)
        pre = _mlp_core(x, dw)
        pats = (pre + _mlp_core(pre, pw)).reshape(T * C, P, D)
        o_ref[0] = _enhance(cls, pats)


def _run_enh(s, d_in, eps_s, dpe, da, pa):
    cspec2 = pl.BlockSpec((T * C, D), lambda i: (0, 0))
    cspec3 = pl.BlockSpec((T * C, P, D), lambda i: (0, 0, 0))
    wspec = [
        pl.BlockSpec((D, D), lambda i: (0, 0)),
        pl.BlockSpec((1, D), lambda i: (0, 0)),
        pl.BlockSpec((D, D), lambda i: (0, 0)),
        pl.BlockSpec((1, D), lambda i: (0, 0)),
        pl.BlockSpec((1, 1), lambda i: (0, 0)),
    ]
    return pl.pallas_call(
        _enh_body,
        grid=(2,),
        in_specs=[cspec2, cspec2, cspec3, cspec3] + wspec + wspec,
        out_specs=pl.BlockSpec((1, T, D), lambda i: (i, 0, 0)),
        out_shape=jax.ShapeDtypeStruct((2, T, D), jnp.float32),
    )(s, d_in, eps_s, dpe, *da, *pa)


def _fw_body(qb, s0_ref, s1_ref, pat_ref, q_ref, w1_ref, b1_ref, w2_ref,
             b2_ref, a_ref, o_ref):
    w = (w1_ref[...], b1_ref[...], w2_ref[...], b2_ref[...], a_ref[0, 0])
    protos = (s0_ref[...] + s1_ref[...]) / float(2 * C)
    x = pat_ref[...].reshape(qb * P, D)
    pf = x + _mlp_core(x, w)
    pats = pf.reshape(qb, P, D)
    qv = q_ref[0]

    dots = lax.dot_general(protos, pf, (((1,), (1,)), ((), ())),
                           preferred_element_type=jnp.float32)
    sq = pf * pf
    ones = jnp.ones((1, D), jnp.float32)
    pn = lax.dot_general(ones, sq, (((1,), (1,)), ((), ())),
                         preferred_element_type=jnp.float32)
    prn = jnp.sqrt(jnp.sum(protos * protos, axis=1, keepdims=True))
    cos = dots / jnp.maximum(jnp.sqrt(pn) * prn, 1e-8)

    rows = jnp.concatenate(
        [lax.slice(cos, (0, q * P), (T, (q + 1) * P)) for q in range(qb)],
        axis=0)
    mx = jnp.max(rows, axis=1, keepdims=True)
    e = jnp.exp(rows - mx)
    w_ = e / jnp.sum(e, axis=1, keepdims=True)
    wm = w_ * _topk_mask(w_, K)
    ws_all = lax.dot_general(wm.reshape(qb, T, P), pats,
                             (((2,), (1,)), ((0,), (0,))),
                             preferred_element_type=jnp.float32)
    for q in range(qb):
        o_ref[:, 0, q, :] = ws_all[q] + 2.0 * qv[q][None, :]


def _run_feature_walk(s0, s1, epq, qv, pa, qb):
    nb = Q // qb
    wspec = [
        pl.BlockSpec((D, D), lambda i: (0, 0)),
        pl.BlockSpec((1, D), lambda i: (0, 0)),
        pl.BlockSpec((D, D), lambda i: (0, 0)),
        pl.BlockSpec((1, D), lambda i: (0, 0)),
        pl.BlockSpec((1, 1), lambda i: (0, 0)),
    ]
    out = pl.pallas_call(
        functools.partial(_fw_body, qb),
        grid=(nb,),
        in_specs=[
            pl.BlockSpec((T, D), lambda i: (0, 0)),
            pl.BlockSpec((T, D), lambda i: (0, 0)),
            pl.BlockSpec((qb, P, D), lambda i: (i, 0, 0)),
            pl.BlockSpec((1, qb, D), lambda i: (i, 0, 0)),
        ] + wspec,
        out_specs=pl.BlockSpec((T, 1, qb, D), lambda i: (0, i, 0, 0)),
        out_shape=jax.ShapeDtypeStruct((T, nb, qb, D), jnp.float32),
    )(s0, s1, epq, qv.reshape(nb, qb, D), *pa)
    return out.reshape(T, Q, D)


def _fold_bn(p, pfx):
    s = 1.0 / jnp.sqrt(jnp.float32(1.0 + 1e-5))
    g1 = s * p[pfx + 'bn1_g']
    w1 = p[pfx + 'fc1_w'].T * g1[None, :]
    b1 = (p[pfx + 'fc1_b'] * g1 + p[pfx + 'bn1_b'])[None, :]
    g2 = s * p[pfx + 'bn2_g']
    w2 = p[pfx + 'fc2_w'].T * g2[None, :]
    b2 = (p[pfx + 'fc2_b'] * g2 + p[pfx + 'bn2_b'])[None, :]
    a = p[pfx + 'prelu'].reshape(1, 1)
    return (w1, b1, w2, b2, a)


def kernel(support_set_vectors, query_set_vectors, dalle_emb_support,
           emb_patch_support, emb_patch_query, dalle_patch_embedding, params):
    s = support_set_vectors.reshape(T * C, D)
    q = query_set_vectors.reshape(Q, D)
    d_in = dalle_emb_support.reshape(T * C, D)

    da = _fold_bn(params, 'da_')
    pa = _fold_bn(params, 'pa_')

    sums = _run_enh(s, d_in, emb_patch_support, dalle_patch_embedding,
                    da, pa)
    sum0, sum1 = sums[0], sums[1]
    protos = (sum0 + sum1) / float(2 * C)

    cls_ws = _run_feature_walk(sum0, sum1, emb_patch_query, q, pa, qb=25)
    return (protos, cls_ws)
